# Initial kernel scaffold; baseline (speedup 1.0000x reference)
import jax
import jax.numpy as jnp
from jax.experimental import pallas as pl


def kernel(x, edge_index, W0, a_s0, a_d0, b0, W1, a_s1, a_d1, b1, W2, a_s2, a_d2, b2):
    raise NotImplementedError("write your pallas kernel here")


# speedup vs baseline: 20.8332x; 20.8332x over previous
import functools

import jax
import jax.numpy as jnp
from jax import lax
from jax.experimental import pallas as pl
from jax.experimental.pallas import tpu as pltpu
from jax.experimental.pallas import tpu_sc as plsc

N = 10000
E = 320000
D = 128
ROW = 144
CHUNK = 80
N_TILES = 32
E_PER_TILE = E // N_TILES
N_PAD = 10240
ROWS_PER_TILE = 640


@functools.lru_cache(maxsize=None)
def _sc_edge_pass(heads):
  mesh = plsc.VectorSubcoreMesh(core_axis_name="c", subcore_axis_name="s")
  n_chunks = E_PER_TILE // CHUNK
  sub = D // heads // 16

  @functools.partial(
      pl.kernel,
      out_type=jax.ShapeDtypeStruct((2, N_PAD, ROW), jnp.float32),
      mesh=mesh,
      scratch_types=[
          pltpu.VMEM((1, CHUNK), jnp.int32),
          pltpu.VMEM((1, CHUNK), jnp.int32),
          pltpu.VMEM((CHUNK, 16), jnp.float32),
          pltpu.VMEM((CHUNK, 16), jnp.float32),
          pltpu.VMEM((CHUNK * heads,), jnp.float32),
          pltpu.VMEM((CHUNK, D), jnp.float32),
          pltpu.VMEM((CHUNK, ROW), jnp.float32),
          pltpu.VMEM_SHARED((N_PAD, ROW), jnp.float32),
      ],
      compiler_params=pltpu.CompilerParams(use_tc_tiling_on_sc=False,
                                           needs_layout_passes=False),
  )
  def edge_pass(edge_hbm, h_hbm, atab_hbm, parts_hbm,
                src_v, dst_v, asr_v, adr_v, eexp_v, rows_v, msg_v, out_acc):
    c = lax.axis_index("c")
    s = lax.axis_index("s")
    wid = c * 16 + s

    def _zero_row(i, _):
      for j in range(ROW // 16):
        msg_v[i, pl.ds(j * 16, 16)] = jnp.zeros((16,), jnp.float32)
      return 0
    lax.fori_loop(0, CHUNK, _zero_row, 0)
    row0 = s * ROWS_PER_TILE
    for k in range(8):
      pltpu.sync_copy(msg_v.at[pl.ds(0, CHUNK)],
                      out_acc.at[pl.ds(row0 + k * CHUNK, CHUNK)])
    plsc.subcore_barrier()

    lanes = jnp.arange(16, dtype=jnp.int32)

    def _chunk(j, _):
      base = wid * E_PER_TILE + j * CHUNK
      pltpu.sync_copy(edge_hbm.at[pl.ds(base, CHUNK)], src_v.at[0])
      pltpu.sync_copy(edge_hbm.at[pl.ds(E + base, CHUNK)], dst_v.at[0])

      pltpu.sync_copy(h_hbm.at[src_v.at[0]], rows_v)
      pltpu.sync_copy(atab_hbm.at[src_v.at[0]], asr_v)
      pltpu.sync_copy(atab_hbm.at[dst_v.at[0]], adr_v)

      for g in range(CHUNK // 16):
        rows16 = g * 16 + lanes
        for hd in range(heads):
          a_s = plsc.load_gather(
              asr_v, [rows16, jnp.full((16,), hd, jnp.int32)])
          a_d = plsc.load_gather(
              adr_v, [rows16, jnp.full((16,), 8 + hd, jnp.int32)])
          e = a_s + a_d
          e = jnp.maximum(e, 0.2 * e)
          ee = jnp.exp(e)
          plsc.store_scatter(eexp_v, [rows16 * heads + hd], ee)

      def _edge(i, _):
        evec = plsc.load_gather(
            eexp_v, [i * heads + jnp.minimum(lanes, heads - 1)])
        evec = jnp.where(lanes < heads, evec, 0.0)
        msg_v[i, pl.ds(D, 16)] = evec
        for hd in range(heads):
          mult = plsc.load_gather(
              eexp_v, [jnp.full((16,), i * heads + hd, jnp.int32)])
          for q in range(sub):
            col = hd * (D // heads) + q * 16
            msg_v[i, pl.ds(col, 16)] = rows_v[i, pl.ds(col, 16)] * mult
        return 0
      lax.fori_loop(0, CHUNK, _edge, 0)

      pltpu.sync_copy(msg_v, out_acc.at[dst_v.at[0]], add=True)
      return 0

    lax.fori_loop(0, n_chunks, _chunk, 0)
    plsc.subcore_barrier()

    for k in range(8):
      r = row0 + k * CHUNK
      pltpu.sync_copy(out_acc.at[pl.ds(r, CHUNK)],
                      parts_hbm.at[c, pl.ds(r, CHUNK)])

  return edge_pass


BLK = 1000


def _tc_first(x_ref, w_ref, h_ref, a_ref):
  acc = jnp.dot(x_ref[...], w_ref[...], preferred_element_type=jnp.float32)
  h_ref[...] = acc[:, :D]
  a_ref[...] = acc[:, D:D + 16]


def _tc_mid(heads_prev, two_h, p0_ref, p1_ref, b_ref, w_ref, h_ref, a_ref):
  s = p0_ref[...] + p1_ref[...]
  num = s[:, :D]
  cph = D // heads_prev
  den = [
      jnp.broadcast_to(s[:, D + hd:D + hd + 1], (BLK, cph)) + 1e-16
      for hd in range(heads_prev)
  ]
  den_b = jnp.concatenate(den, axis=1)
  f = num / den_b + b_ref[...]
  f = jnp.where(f > 0, f, jnp.exp(jnp.minimum(f, 0.0)) - 1.0)
  acc = jnp.dot(f, w_ref[...], preferred_element_type=jnp.float32)
  h_ref[...] = acc[:, :D]
  a_ref[...] = acc[:, D:D + two_h]


def _tc_final(p0_ref, p1_ref, b_ref, out_ref):
  s = p0_ref[...] + p1_ref[...]
  den = jnp.broadcast_to(s[:, D:D + 1], (BLK, D)) + 1e-16
  out_ref[...] = s[:, :D] / den + b_ref[...]


def _dense_stage(body, inputs, two_h, n_extra_w):
  grid = (N // BLK,)
  in_specs = []
  for _ in range(n_extra_w):
    in_specs.append(pl.BlockSpec((BLK, ROW), lambda i: (i, 0)))
  if n_extra_w:
    in_specs.append(pl.BlockSpec((1, D), lambda i: (0, 0)))
  else:
    in_specs.append(pl.BlockSpec((BLK, D), lambda i: (i, 0)))
  in_specs.append(pl.BlockSpec((D, 256), lambda i: (0, 0)))
  return pl.pallas_call(
      body,
      grid=grid,
      in_specs=in_specs,
      out_specs=[
          pl.BlockSpec((BLK, D), lambda i: (i, 0)),
          pl.BlockSpec((BLK, two_h), lambda i: (i, 0)),
      ],
      out_shape=[
          jax.ShapeDtypeStruct((N, D), jnp.float32),
          jax.ShapeDtypeStruct((N, two_h), jnp.float32),
      ],
  )(*inputs)


def _final_stage(p0, p1, b):
  return pl.pallas_call(
      _tc_final,
      grid=(N // BLK,),
      in_specs=[
          pl.BlockSpec((BLK, ROW), lambda i: (i, 0)),
          pl.BlockSpec((BLK, ROW), lambda i: (i, 0)),
          pl.BlockSpec((1, D), lambda i: (0, 0)),
      ],
      out_specs=pl.BlockSpec((BLK, D), lambda i: (i, 0)),
      out_shape=jax.ShapeDtypeStruct((N, D), jnp.float32),
  )(p0, p1, b)


def _wcat(W, a_s, a_d, heads, out_ch):
  A_s = jnp.zeros((heads * out_ch, 8), W.dtype)
  A_d = jnp.zeros((heads * out_ch, 8), W.dtype)
  for hd in range(heads):
    A_s = A_s.at[hd * out_ch:(hd + 1) * out_ch, hd].set(a_s[hd])
    A_d = A_d.at[hd * out_ch:(hd + 1) * out_ch, hd].set(a_d[hd])
  cat = jnp.concatenate([W, W @ A_s, W @ A_d], axis=1)
  pad = jnp.zeros((W.shape[0], 256 - cat.shape[1]), W.dtype)
  return jnp.concatenate([cat, pad], axis=1)


def kernel(x, edge_index, W0, a_s0, a_d0, b0, W1, a_s1, a_d1, b1,
           W2, a_s2, a_d2, b2):
  edge_flat = edge_index.reshape(-1)
  w0 = _wcat(W0, a_s0, a_d0, 4, 32)
  w1 = _wcat(W1, a_s1, a_d1, 4, 32)
  w2 = _wcat(W2, a_s2, a_d2, 1, D)

  h, atab = _dense_stage(_tc_first, [x, w0], 16, 0)
  parts = _sc_edge_pass(4)(edge_flat, h, atab)

  h, atab = _dense_stage(
      functools.partial(_tc_mid, 4, 16),
      [parts[0], parts[1], b0[None, :], w1], 16, 2)
  parts = _sc_edge_pass(4)(edge_flat, h, atab)

  h, atab = _dense_stage(
      functools.partial(_tc_mid, 4, 16),
      [parts[0], parts[1], b1[None, :], w2], 16, 2)
  parts = _sc_edge_pass(1)(edge_flat, h, atab)

  return _final_stage(parts[0], parts[1], b2[None, :])


# speedup vs baseline: 45.6456x; 2.1910x over previous
import functools

import jax
import jax.numpy as jnp
from jax import lax
from jax.experimental import pallas as pl
from jax.experimental.pallas import tpu as pltpu
from jax.experimental.pallas import tpu_sc as plsc

N = 10000
E = 320000
D = 128
ROW = 144
CHUNK = 80
N_TILES = 32
E_PER_TILE = E // N_TILES
N_PAD = 10240
ROWS_PER_TILE = 640


@functools.lru_cache(maxsize=None)
def _sc_edge_pass(heads):
  mesh = plsc.VectorSubcoreMesh(core_axis_name="c", subcore_axis_name="s")
  n_chunks = E_PER_TILE // CHUNK
  sub = D // heads // 16

  @functools.partial(
      pl.kernel,
      out_type=jax.ShapeDtypeStruct((2, N_PAD, ROW), jnp.float32),
      mesh=mesh,
      scratch_types=[
          pltpu.VMEM((1, CHUNK), jnp.int32),
          pltpu.VMEM((1, CHUNK), jnp.int32),
          pltpu.VMEM((CHUNK, 16), jnp.float32),
          pltpu.VMEM((CHUNK, 16), jnp.float32),
          pltpu.VMEM((CHUNK * heads,), jnp.float32),
          pltpu.VMEM((CHUNK, D), jnp.float32),
          pltpu.VMEM((CHUNK, ROW), jnp.float32),
          pltpu.VMEM_SHARED((N_PAD, ROW), jnp.float32),
          pltpu.SemaphoreType.DMA,
          pltpu.SemaphoreType.DMA,
      ],
      compiler_params=pltpu.CompilerParams(use_tc_tiling_on_sc=False,
                                           needs_layout_passes=False),
  )
  def edge_pass(edge_hbm, h_hbm, atab_hbm, parts_hbm,
                src_v, dst_v, asr_v, adr_v, eexp_v, rows_v, msg_v, out_acc,
                sem_i, sem_g):
    c = lax.axis_index("c")
    s = lax.axis_index("s")
    wid = c * 16 + s

    def _zero_row(i, _):
      for j in range(ROW // 16):
        msg_v[i, pl.ds(j * 16, 16)] = jnp.zeros((16,), jnp.float32)
      return 0
    lax.fori_loop(0, CHUNK, _zero_row, 0)
    row0 = s * ROWS_PER_TILE
    for k in range(8):
      pltpu.sync_copy(msg_v.at[pl.ds(0, CHUNK)],
                      out_acc.at[pl.ds(row0 + k * CHUNK, CHUNK)])
    plsc.subcore_barrier()

    lanes = jnp.arange(16, dtype=jnp.int32)

    def _chunk(j, _):
      base = wid * E_PER_TILE + j * CHUNK
      i1 = pltpu.async_copy(edge_hbm.at[pl.ds(base, CHUNK)], src_v.at[0],
                            sem_i)
      i2 = pltpu.async_copy(edge_hbm.at[pl.ds(E + base, CHUNK)], dst_v.at[0],
                            sem_i)
      i1.wait()
      i2.wait()

      g1 = pltpu.async_copy(h_hbm.at[src_v.at[0]], rows_v, sem_g)
      g2 = pltpu.async_copy(atab_hbm.at[src_v.at[0]], asr_v, sem_g)
      g3 = pltpu.async_copy(atab_hbm.at[dst_v.at[0]], adr_v, sem_g)
      g1.wait()
      g2.wait()
      g3.wait()

      for g in range(CHUNK // 16):
        rows16 = g * 16 + lanes
        for hd in range(heads):
          a_s = plsc.load_gather(
              asr_v, [rows16, jnp.full((16,), hd, jnp.int32)])
          a_d = plsc.load_gather(
              adr_v, [rows16, jnp.full((16,), 8 + hd, jnp.int32)])
          e = a_s + a_d
          e = jnp.maximum(e, 0.2 * e)
          ee = jnp.exp(e)
          plsc.store_scatter(eexp_v, [rows16 * heads + hd], ee)

      @plsc.parallel_loop(0, CHUNK, unroll=4)
      def _edge(i):
        evec = plsc.load_gather(
            eexp_v, [i * heads + jnp.minimum(lanes, heads - 1)])
        evec = jnp.where(lanes < heads, evec, 0.0)
        msg_v[i, pl.ds(D, 16)] = evec
        for hd in range(heads):
          mult = plsc.load_gather(
              eexp_v, [jnp.full((16,), i * heads + hd, jnp.int32)])
          for q in range(sub):
            col = hd * (D // heads) + q * 16
            msg_v[i, pl.ds(col, 16)] = rows_v[i, pl.ds(col, 16)] * mult

      pltpu.sync_copy(msg_v, out_acc.at[dst_v.at[0]], add=True)
      return 0

    lax.fori_loop(0, n_chunks, _chunk, 0)
    plsc.subcore_barrier()

    for k in range(8):
      r = row0 + k * CHUNK
      pltpu.sync_copy(out_acc.at[pl.ds(r, CHUNK)],
                      parts_hbm.at[c, pl.ds(r, CHUNK)])

  return edge_pass


BLK = 1000


def _tc_first(x_ref, w_ref, h_ref, a_ref):
  acc = jnp.dot(x_ref[...], w_ref[...], preferred_element_type=jnp.float32)
  h_ref[...] = acc[:, :D]
  a_ref[...] = acc[:, D:D + 16]


def _tc_mid(heads_prev, two_h, p0_ref, p1_ref, b_ref, w_ref, h_ref, a_ref):
  s = p0_ref[...] + p1_ref[...]
  num = s[:, :D]
  cph = D // heads_prev
  den = [
      jnp.broadcast_to(s[:, D + hd:D + hd + 1], (BLK, cph)) + 1e-16
      for hd in range(heads_prev)
  ]
  den_b = jnp.concatenate(den, axis=1)
  f = num / den_b + b_ref[...]
  f = jnp.where(f > 0, f, jnp.exp(jnp.minimum(f, 0.0)) - 1.0)
  acc = jnp.dot(f, w_ref[...], preferred_element_type=jnp.float32)
  h_ref[...] = acc[:, :D]
  a_ref[...] = acc[:, D:D + two_h]


def _tc_final(p0_ref, p1_ref, b_ref, out_ref):
  s = p0_ref[...] + p1_ref[...]
  den = jnp.broadcast_to(s[:, D:D + 1], (BLK, D)) + 1e-16
  out_ref[...] = s[:, :D] / den + b_ref[...]


def _dense_stage(body, inputs, two_h, n_extra_w):
  grid = (N // BLK,)
  in_specs = []
  for _ in range(n_extra_w):
    in_specs.append(pl.BlockSpec((BLK, ROW), lambda i: (i, 0)))
  if n_extra_w:
    in_specs.append(pl.BlockSpec((1, D), lambda i: (0, 0)))
  else:
    in_specs.append(pl.BlockSpec((BLK, D), lambda i: (i, 0)))
  in_specs.append(pl.BlockSpec((D, 256), lambda i: (0, 0)))
  return pl.pallas_call(
      body,
      grid=grid,
      in_specs=in_specs,
      out_specs=[
          pl.BlockSpec((BLK, D), lambda i: (i, 0)),
          pl.BlockSpec((BLK, two_h), lambda i: (i, 0)),
      ],
      out_shape=[
          jax.ShapeDtypeStruct((N, D), jnp.float32),
          jax.ShapeDtypeStruct((N, two_h), jnp.float32),
      ],
  )(*inputs)


def _final_stage(p0, p1, b):
  return pl.pallas_call(
      _tc_final,
      grid=(N // BLK,),
      in_specs=[
          pl.BlockSpec((BLK, ROW), lambda i: (i, 0)),
          pl.BlockSpec((BLK, ROW), lambda i: (i, 0)),
          pl.BlockSpec((1, D), lambda i: (0, 0)),
      ],
      out_specs=pl.BlockSpec((BLK, D), lambda i: (i, 0)),
      out_shape=jax.ShapeDtypeStruct((N, D), jnp.float32),
  )(p0, p1, b)


def _wcat(W, a_s, a_d, heads, out_ch):
  A_s = jnp.zeros((heads * out_ch, 8), W.dtype)
  A_d = jnp.zeros((heads * out_ch, 8), W.dtype)
  for hd in range(heads):
    A_s = A_s.at[hd * out_ch:(hd + 1) * out_ch, hd].set(a_s[hd])
    A_d = A_d.at[hd * out_ch:(hd + 1) * out_ch, hd].set(a_d[hd])
  cat = jnp.concatenate([W, W @ A_s, W @ A_d], axis=1)
  pad = jnp.zeros((W.shape[0], 256 - cat.shape[1]), W.dtype)
  return jnp.concatenate([cat, pad], axis=1)


def kernel(x, edge_index, W0, a_s0, a_d0, b0, W1, a_s1, a_d1, b1,
           W2, a_s2, a_d2, b2):
  edge_flat = edge_index.reshape(-1)
  w0 = _wcat(W0, a_s0, a_d0, 4, 32)
  w1 = _wcat(W1, a_s1, a_d1, 4, 32)
  w2 = _wcat(W2, a_s2, a_d2, 1, D)

  h, atab = _dense_stage(_tc_first, [x, w0], 16, 0)
  parts = _sc_edge_pass(4)(edge_flat, h, atab)

  h, atab = _dense_stage(
      functools.partial(_tc_mid, 4, 16),
      [parts[0], parts[1], b0[None, :], w1], 16, 2)
  parts = _sc_edge_pass(4)(edge_flat, h, atab)

  h, atab = _dense_stage(
      functools.partial(_tc_mid, 4, 16),
      [parts[0], parts[1], b1[None, :], w2], 16, 2)
  parts = _sc_edge_pass(1)(edge_flat, h, atab)

  return _final_stage(parts[0], parts[1], b2[None, :])


# speedup vs baseline: 52.0749x; 1.1409x over previous
import functools

import jax
import jax.numpy as jnp
from jax import lax
from jax.experimental import pallas as pl
from jax.experimental.pallas import tpu as pltpu
from jax.experimental.pallas import tpu_sc as plsc

N = 10000
E = 320000
D = 128
ROW = 144
CHUNK = 80
N_TILES = 32
E_PER_TILE = E // N_TILES
N_PAD = 10240
ROWS_PER_TILE = 640


@functools.lru_cache(maxsize=None)
def _sc_edge_pass(heads):
  mesh = plsc.VectorSubcoreMesh(core_axis_name="c", subcore_axis_name="s")
  n_chunks = E_PER_TILE // CHUNK
  sub = D // heads // 16

  @functools.partial(
      pl.kernel,
      out_type=jax.ShapeDtypeStruct((2, N_PAD, ROW), jnp.float32),
      mesh=mesh,
      scratch_types=[
          pltpu.VMEM((1, CHUNK), jnp.int32),
          pltpu.VMEM((1, CHUNK), jnp.int32),
          pltpu.VMEM((1, CHUNK), jnp.int32),
          pltpu.VMEM((CHUNK, 16), jnp.float32),
          pltpu.VMEM((CHUNK, 16), jnp.float32),
          pltpu.VMEM((CHUNK * heads,), jnp.float32),
          pltpu.VMEM((CHUNK, D), jnp.float32),
          pltpu.VMEM((CHUNK, ROW), jnp.float32),
          pltpu.VMEM_SHARED((N_PAD, ROW), jnp.float32),
          pltpu.SemaphoreType.DMA,
          pltpu.SemaphoreType.DMA,
          pltpu.SemaphoreType.DMA,
      ],
      compiler_params=pltpu.CompilerParams(use_tc_tiling_on_sc=False,
                                           needs_layout_passes=False),
  )
  def edge_pass(edge_hbm, h_hbm, atab_hbm, parts_hbm,
                src_v, dst_v, dst_s, asr_v, adr_v, eexp_v, rows_v, msg_v,
                out_acc, sem_i, sem_g, sem_s):
    c = lax.axis_index("c")
    s = lax.axis_index("s")
    wid = c * 16 + s

    def _zero_row(i, _):
      for j in range(ROW // 16):
        msg_v[i, pl.ds(j * 16, 16)] = jnp.zeros((16,), jnp.float32)
      return 0
    lax.fori_loop(0, CHUNK, _zero_row, 0)
    row0 = s * ROWS_PER_TILE
    for k in range(8):
      pltpu.sync_copy(msg_v.at[pl.ds(0, CHUNK)],
                      out_acc.at[pl.ds(row0 + k * CHUNK, CHUNK)])
    plsc.subcore_barrier()

    lanes = jnp.arange(16, dtype=jnp.int32)

    def _chunk(j, _):
      base = wid * E_PER_TILE + j * CHUNK
      i1 = pltpu.async_copy(edge_hbm.at[pl.ds(base, CHUNK)], src_v.at[0],
                            sem_i)
      i2 = pltpu.async_copy(edge_hbm.at[pl.ds(E + base, CHUNK)], dst_v.at[0],
                            sem_i)
      i1.wait()
      i2.wait()

      g1 = pltpu.async_copy(h_hbm.at[src_v.at[0]], rows_v, sem_g)
      g2 = pltpu.async_copy(atab_hbm.at[src_v.at[0]], asr_v, sem_g)
      g3 = pltpu.async_copy(atab_hbm.at[dst_v.at[0]], adr_v, sem_g)
      g1.wait()
      g2.wait()
      g3.wait()

      for g in range(CHUNK // 16):
        rows16 = g * 16 + lanes
        for hd in range(heads):
          a_s = plsc.load_gather(
              asr_v, [rows16, jnp.full((16,), hd, jnp.int32)])
          a_d = plsc.load_gather(
              adr_v, [rows16, jnp.full((16,), 8 + hd, jnp.int32)])
          e = a_s + a_d
          e = jnp.maximum(e, 0.2 * e)
          ee = jnp.exp(e)
          plsc.store_scatter(eexp_v, [rows16 * heads + hd], ee)

      @pl.when(j >= 1)
      def _drain_prev():
        pltpu.make_async_copy(msg_v, out_acc.at[dst_s.at[0]], sem_s).wait()

      @plsc.parallel_loop(0, CHUNK, unroll=4)
      def _edge(i):
        evec = plsc.load_gather(
            eexp_v, [i * heads + jnp.minimum(lanes, heads - 1)])
        evec = jnp.where(lanes < heads, evec, 0.0)
        msg_v[i, pl.ds(D, 16)] = evec
        for hd in range(heads):
          mult = plsc.load_gather(
              eexp_v, [jnp.full((16,), i * heads + hd, jnp.int32)])
          for q in range(sub):
            col = hd * (D // heads) + q * 16
            msg_v[i, pl.ds(col, 16)] = rows_v[i, pl.ds(col, 16)] * mult

      for g in range(CHUNK // 16):
        dst_s[0, pl.ds(g * 16, 16)] = dst_v[0, pl.ds(g * 16, 16)]

      pltpu.async_copy(msg_v, out_acc.at[dst_s.at[0]], sem_s, add=True)
      return 0

    lax.fori_loop(0, n_chunks, _chunk, 0)
    pltpu.make_async_copy(msg_v, out_acc.at[dst_s.at[0]], sem_s).wait()
    plsc.subcore_barrier()

    for k in range(8):
      r = row0 + k * CHUNK
      pltpu.sync_copy(out_acc.at[pl.ds(r, CHUNK)],
                      parts_hbm.at[c, pl.ds(r, CHUNK)])

  return edge_pass


BLK = 1000


def _tc_first(x_ref, w_ref, h_ref, a_ref):
  acc = jnp.dot(x_ref[...], w_ref[...], preferred_element_type=jnp.float32)
  h_ref[...] = acc[:, :D]
  a_ref[...] = acc[:, D:D + 16]


def _tc_mid(heads_prev, two_h, p0_ref, p1_ref, b_ref, w_ref, h_ref, a_ref):
  s = p0_ref[...] + p1_ref[...]
  num = s[:, :D]
  cph = D // heads_prev
  den = [
      jnp.broadcast_to(s[:, D + hd:D + hd + 1], (BLK, cph)) + 1e-16
      for hd in range(heads_prev)
  ]
  den_b = jnp.concatenate(den, axis=1)
  f = num / den_b + b_ref[...]
  f = jnp.where(f > 0, f, jnp.exp(jnp.minimum(f, 0.0)) - 1.0)
  acc = jnp.dot(f, w_ref[...], preferred_element_type=jnp.float32)
  h_ref[...] = acc[:, :D]
  a_ref[...] = acc[:, D:D + two_h]


def _tc_final(p0_ref, p1_ref, b_ref, out_ref):
  s = p0_ref[...] + p1_ref[...]
  den = jnp.broadcast_to(s[:, D:D + 1], (BLK, D)) + 1e-16
  out_ref[...] = s[:, :D] / den + b_ref[...]


def _dense_stage(body, inputs, two_h, n_extra_w):
  grid = (N // BLK,)
  in_specs = []
  for _ in range(n_extra_w):
    in_specs.append(pl.BlockSpec((BLK, ROW), lambda i: (i, 0)))
  if n_extra_w:
    in_specs.append(pl.BlockSpec((1, D), lambda i: (0, 0)))
  else:
    in_specs.append(pl.BlockSpec((BLK, D), lambda i: (i, 0)))
  in_specs.append(pl.BlockSpec((D, 256), lambda i: (0, 0)))
  return pl.pallas_call(
      body,
      grid=grid,
      in_specs=in_specs,
      out_specs=[
          pl.BlockSpec((BLK, D), lambda i: (i, 0)),
          pl.BlockSpec((BLK, two_h), lambda i: (i, 0)),
      ],
      out_shape=[
          jax.ShapeDtypeStruct((N, D), jnp.float32),
          jax.ShapeDtypeStruct((N, two_h), jnp.float32),
      ],
  )(*inputs)


def _final_stage(p0, p1, b):
  return pl.pallas_call(
      _tc_final,
      grid=(N // BLK,),
      in_specs=[
          pl.BlockSpec((BLK, ROW), lambda i: (i, 0)),
          pl.BlockSpec((BLK, ROW), lambda i: (i, 0)),
          pl.BlockSpec((1, D), lambda i: (0, 0)),
      ],
      out_specs=pl.BlockSpec((BLK, D), lambda i: (i, 0)),
      out_shape=jax.ShapeDtypeStruct((N, D), jnp.float32),
  )(p0, p1, b)


def _wcat(W, a_s, a_d, heads, out_ch):
  A_s = jnp.zeros((heads * out_ch, 8), W.dtype)
  A_d = jnp.zeros((heads * out_ch, 8), W.dtype)
  for hd in range(heads):
    A_s = A_s.at[hd * out_ch:(hd + 1) * out_ch, hd].set(a_s[hd])
    A_d = A_d.at[hd * out_ch:(hd + 1) * out_ch, hd].set(a_d[hd])
  cat = jnp.concatenate([W, W @ A_s, W @ A_d], axis=1)
  pad = jnp.zeros((W.shape[0], 256 - cat.shape[1]), W.dtype)
  return jnp.concatenate([cat, pad], axis=1)


def kernel(x, edge_index, W0, a_s0, a_d0, b0, W1, a_s1, a_d1, b1,
           W2, a_s2, a_d2, b2):
  edge_flat = edge_index.reshape(-1)
  w0 = _wcat(W0, a_s0, a_d0, 4, 32)
  w1 = _wcat(W1, a_s1, a_d1, 4, 32)
  w2 = _wcat(W2, a_s2, a_d2, 1, D)

  h, atab = _dense_stage(_tc_first, [x, w0], 16, 0)
  parts = _sc_edge_pass(4)(edge_flat, h, atab)

  h, atab = _dense_stage(
      functools.partial(_tc_mid, 4, 16),
      [parts[0], parts[1], b0[None, :], w1], 16, 2)
  parts = _sc_edge_pass(4)(edge_flat, h, atab)

  h, atab = _dense_stage(
      functools.partial(_tc_mid, 4, 16),
      [parts[0], parts[1], b1[None, :], w2], 16, 2)
  parts = _sc_edge_pass(1)(edge_flat, h, atab)

  return _final_stage(parts[0], parts[1], b2[None, :])


# speedup vs baseline: 80.5091x; 1.5460x over previous
import functools

import jax
import jax.numpy as jnp
from jax import lax
from jax.experimental import pallas as pl
from jax.experimental.pallas import tpu as pltpu
from jax.experimental.pallas import tpu_sc as plsc

N = 10000
E = 320000
D = 128
ROW = 144
CHUNK = 80
N_TILES = 32
E_PER_TILE = E // N_TILES
N_PAD = 10240
ROWS_PER_TILE = 640


@functools.lru_cache(maxsize=None)
def _sc_edge_pass(heads):
  mesh = plsc.VectorSubcoreMesh(core_axis_name="c", subcore_axis_name="s")
  n_chunks = E_PER_TILE // CHUNK
  n_super = (n_chunks - 1) // 2
  sub = D // heads // 16

  @functools.partial(
      pl.kernel,
      out_type=jax.ShapeDtypeStruct((2, N_PAD, ROW), jnp.float32),
      mesh=mesh,
      scratch_types=[
          pltpu.VMEM((1, CHUNK), jnp.int32),
          pltpu.VMEM((1, CHUNK), jnp.int32),
          pltpu.VMEM((1, CHUNK), jnp.int32),
          pltpu.VMEM((1, CHUNK), jnp.int32),
          pltpu.VMEM((1, CHUNK), jnp.int32),
          pltpu.VMEM((CHUNK, 16), jnp.float32),
          pltpu.VMEM((CHUNK, 16), jnp.float32),
          pltpu.VMEM((CHUNK, 16), jnp.float32),
          pltpu.VMEM((CHUNK, 16), jnp.float32),
          pltpu.VMEM((CHUNK, D), jnp.float32),
          pltpu.VMEM((CHUNK, D), jnp.float32),
          pltpu.VMEM((CHUNK * heads,), jnp.float32),
          pltpu.VMEM((CHUNK, ROW), jnp.float32),
          pltpu.VMEM_SHARED((N_PAD, ROW), jnp.float32),
          pltpu.SemaphoreType.DMA,
          pltpu.SemaphoreType.DMA,
          pltpu.SemaphoreType.DMA,
          pltpu.SemaphoreType.DMA,
          pltpu.SemaphoreType.DMA,
      ],
      compiler_params=pltpu.CompilerParams(use_tc_tiling_on_sc=False,
                                           needs_layout_passes=False),
  )
  def edge_pass(edge_hbm, h_hbm, atab_hbm, parts_hbm,
                src0, dst0, src1, dst1, dst_s, asr0, adr0, asr1, adr1,
                rows0, rows1, eexp_v, msg_v, out_acc,
                sem_i0, sem_i1, sem_g0, sem_g1, sem_s):
    c = lax.axis_index("c")
    s = lax.axis_index("s")
    wid = c * 16 + s
    bufs = [(src0, dst0, asr0, adr0, rows0, sem_i0, sem_g0),
            (src1, dst1, asr1, adr1, rows1, sem_i1, sem_g1)]
    lanes = jnp.arange(16, dtype=jnp.int32)

    def _zero_row(i, _):
      for j in range(ROW // 16):
        msg_v[i, pl.ds(j * 16, 16)] = jnp.zeros((16,), jnp.float32)
      return 0
    lax.fori_loop(0, CHUNK, _zero_row, 0)
    row0 = s * ROWS_PER_TILE
    for k in range(8):
      pltpu.sync_copy(msg_v.at[pl.ds(0, CHUNK)],
                      out_acc.at[pl.ds(row0 + k * CHUNK, CHUNK)])
    plsc.subcore_barrier()

    def start_idx(cix, b):
      srcb, dstb, _, _, _, semib, _ = bufs[b]
      base = wid * E_PER_TILE + cix * CHUNK
      pltpu.async_copy(edge_hbm.at[pl.ds(base, CHUNK)], srcb.at[0], semib)
      pltpu.async_copy(edge_hbm.at[pl.ds(E + base, CHUNK)], dstb.at[0],
                       semib)

    def wait_idx(b):
      srcb, dstb, _, _, _, semib, _ = bufs[b]
      pltpu.make_async_copy(edge_hbm.at[pl.ds(0, CHUNK)], srcb.at[0],
                            semib).wait()
      pltpu.make_async_copy(edge_hbm.at[pl.ds(0, CHUNK)], dstb.at[0],
                            semib).wait()

    def start_gathers(b):
      srcb, dstb, asrb, adrb, rowsb, _, semgb = bufs[b]
      pltpu.async_copy(h_hbm.at[srcb.at[0]], rowsb, semgb)
      pltpu.async_copy(atab_hbm.at[srcb.at[0]], asrb, semgb)
      pltpu.async_copy(atab_hbm.at[dstb.at[0]], adrb, semgb)

    def wait_gathers(b):
      srcb, dstb, asrb, adrb, rowsb, _, semgb = bufs[b]
      pltpu.make_async_copy(h_hbm.at[srcb.at[0]], rowsb, semgb).wait()
      pltpu.make_async_copy(atab_hbm.at[srcb.at[0]], asrb, semgb).wait()
      pltpu.make_async_copy(atab_hbm.at[dstb.at[0]], adrb, semgb).wait()

    def scatter_wait():
      pltpu.make_async_copy(msg_v, out_acc.at[dst_s.at[0]], sem_s).wait()

    def compute_and_scatter(b):
      srcb, dstb, asrb, adrb, rowsb, _, _ = bufs[b]
      for g in range(CHUNK // 16):
        rows16 = g * 16 + lanes
        for hd in range(heads):
          a_s = plsc.load_gather(
              asrb, [rows16, jnp.full((16,), hd, jnp.int32)])
          a_d = plsc.load_gather(
              adrb, [rows16, jnp.full((16,), 8 + hd, jnp.int32)])
          e = a_s + a_d
          e = jnp.maximum(e, 0.2 * e)
          ee = jnp.exp(e)
          plsc.store_scatter(eexp_v, [rows16 * heads + hd], ee)

      @plsc.parallel_loop(0, CHUNK, unroll=4)
      def _edge(i):
        evec = plsc.load_gather(
            eexp_v, [i * heads + jnp.minimum(lanes, heads - 1)])
        evec = jnp.where(lanes < heads, evec, 0.0)
        msg_v[i, pl.ds(D, 16)] = evec
        for hd in range(heads):
          mult = plsc.load_gather(
              eexp_v, [jnp.full((16,), i * heads + hd, jnp.int32)])
          for q in range(sub):
            col = hd * (D // heads) + q * 16
            msg_v[i, pl.ds(col, 16)] = rowsb[i, pl.ds(col, 16)] * mult

      for g in range(CHUNK // 16):
        dst_s[0, pl.ds(g * 16, 16)] = dstb[0, pl.ds(g * 16, 16)]
      pltpu.async_copy(msg_v, out_acc.at[dst_s.at[0]], sem_s, add=True)

    start_idx(0, 0)
    wait_idx(0)
    start_gathers(0)
    start_idx(1, 1)

    def _super(k, _):
      wait_gathers(0)
      @pl.when(k >= 1)
      def _():
        scatter_wait()
      wait_idx(1)
      start_gathers(1)
      compute_and_scatter(0)
      start_idx(2 * k + 2, 0)
      wait_gathers(1)
      scatter_wait()
      wait_idx(0)
      start_gathers(0)
      compute_and_scatter(1)
      @pl.when(k <= n_super - 2)
      def _():
        start_idx(2 * k + 3, 1)
      return 0

    lax.fori_loop(0, n_super, _super, 0)

    wait_gathers(0)
    scatter_wait()
    compute_and_scatter(0)
    scatter_wait()
    plsc.subcore_barrier()

    for k in range(8):
      r = row0 + k * CHUNK
      pltpu.sync_copy(out_acc.at[pl.ds(r, CHUNK)],
                      parts_hbm.at[c, pl.ds(r, CHUNK)])

  return edge_pass


BLK = 1000


def _tc_first(x_ref, w_ref, h_ref, a_ref):
  acc = jnp.dot(x_ref[...], w_ref[...], preferred_element_type=jnp.float32)
  h_ref[...] = acc[:, :D]
  a_ref[...] = acc[:, D:D + 16]


def _tc_mid(heads_prev, two_h, p0_ref, p1_ref, b_ref, w_ref, h_ref, a_ref):
  s = p0_ref[...] + p1_ref[...]
  num = s[:, :D]
  cph = D // heads_prev
  den = [
      jnp.broadcast_to(s[:, D + hd:D + hd + 1], (BLK, cph)) + 1e-16
      for hd in range(heads_prev)
  ]
  den_b = jnp.concatenate(den, axis=1)
  f = num / den_b + b_ref[...]
  f = jnp.where(f > 0, f, jnp.exp(jnp.minimum(f, 0.0)) - 1.0)
  acc = jnp.dot(f, w_ref[...], preferred_element_type=jnp.float32)
  h_ref[...] = acc[:, :D]
  a_ref[...] = acc[:, D:D + two_h]


def _tc_final(p0_ref, p1_ref, b_ref, out_ref):
  s = p0_ref[...] + p1_ref[...]
  den = jnp.broadcast_to(s[:, D:D + 1], (BLK, D)) + 1e-16
  out_ref[...] = s[:, :D] / den + b_ref[...]


def _dense_stage(body, inputs, two_h, n_extra_w):
  grid = (N // BLK,)
  in_specs = []
  for _ in range(n_extra_w):
    in_specs.append(pl.BlockSpec((BLK, ROW), lambda i: (i, 0)))
  if n_extra_w:
    in_specs.append(pl.BlockSpec((1, D), lambda i: (0, 0)))
  else:
    in_specs.append(pl.BlockSpec((BLK, D), lambda i: (i, 0)))
  in_specs.append(pl.BlockSpec((D, 256), lambda i: (0, 0)))
  return pl.pallas_call(
      body,
      grid=grid,
      in_specs=in_specs,
      out_specs=[
          pl.BlockSpec((BLK, D), lambda i: (i, 0)),
          pl.BlockSpec((BLK, two_h), lambda i: (i, 0)),
      ],
      out_shape=[
          jax.ShapeDtypeStruct((N, D), jnp.float32),
          jax.ShapeDtypeStruct((N, two_h), jnp.float32),
      ],
  )(*inputs)


def _final_stage(p0, p1, b):
  return pl.pallas_call(
      _tc_final,
      grid=(N // BLK,),
      in_specs=[
          pl.BlockSpec((BLK, ROW), lambda i: (i, 0)),
          pl.BlockSpec((BLK, ROW), lambda i: (i, 0)),
          pl.BlockSpec((1, D), lambda i: (0, 0)),
      ],
      out_specs=pl.BlockSpec((BLK, D), lambda i: (i, 0)),
      out_shape=jax.ShapeDtypeStruct((N, D), jnp.float32),
  )(p0, p1, b)


def _wcat(W, a_s, a_d, heads, out_ch):
  A_s = jnp.zeros((heads * out_ch, 8), W.dtype)
  A_d = jnp.zeros((heads * out_ch, 8), W.dtype)
  for hd in range(heads):
    A_s = A_s.at[hd * out_ch:(hd + 1) * out_ch, hd].set(a_s[hd])
    A_d = A_d.at[hd * out_ch:(hd + 1) * out_ch, hd].set(a_d[hd])
  cat = jnp.concatenate([W, W @ A_s, W @ A_d], axis=1)
  pad = jnp.zeros((W.shape[0], 256 - cat.shape[1]), W.dtype)
  return jnp.concatenate([cat, pad], axis=1)


def kernel(x, edge_index, W0, a_s0, a_d0, b0, W1, a_s1, a_d1, b1,
           W2, a_s2, a_d2, b2):
  edge_flat = edge_index.reshape(-1)
  w0 = _wcat(W0, a_s0, a_d0, 4, 32)
  w1 = _wcat(W1, a_s1, a_d1, 4, 32)
  w2 = _wcat(W2, a_s2, a_d2, 1, D)

  h, atab = _dense_stage(_tc_first, [x, w0], 16, 0)
  parts = _sc_edge_pass(4)(edge_flat, h, atab)

  h, atab = _dense_stage(
      functools.partial(_tc_mid, 4, 16),
      [parts[0], parts[1], b0[None, :], w1], 16, 2)
  parts = _sc_edge_pass(4)(edge_flat, h, atab)

  h, atab = _dense_stage(
      functools.partial(_tc_mid, 4, 16),
      [parts[0], parts[1], b1[None, :], w2], 16, 2)
  parts = _sc_edge_pass(1)(edge_flat, h, atab)

  return _final_stage(parts[0], parts[1], b2[None, :])
